# baseline (speedup 1.0000x reference)
import functools

import jax
import jax.numpy as jnp
from jax import lax
from jax.experimental import pallas as pl
from jax.experimental.pallas import tpu as pltpu
from jax.experimental.pallas import tpu_sc as plsc

N_NODES = 100000
ROW = 512
BATCH = 16384
L = 16
NW = 32
RPW_MIN = 3120
RPW_PAD = 3136
LIST_LEN = RPW_PAD + L


@functools.cache
def _build_sc_scatter():
    mesh = plsc.VectorSubcoreMesh(
        core_axis_name="c", subcore_axis_name="s", num_cores=2, num_subcores=16
    )
    return pl.kernel(
        _sc_scatter_body,
        out_type=jax.ShapeDtypeStruct((N_NODES, ROW), jnp.float32),
        mesh=mesh,
        compiler_params=pltpu.CompilerParams(needs_layout_passes=False),
        scratch_types=[
            pltpu.VMEM((BATCH,), jnp.int32),
            pltpu.VMEM((RPW_PAD,), jnp.int32),
            pltpu.VMEM((LIST_LEN,), jnp.int32),
            pltpu.VMEM((LIST_LEN,), jnp.int32),
            pltpu.VMEM((L, ROW), jnp.float32),
            pltpu.SemaphoreType.DMA,
            pltpu.SemaphoreType.DMA,
            pltpu.SemaphoreType.DMA,
            pltpu.SemaphoreType.DMA,
        ],
    )


def _sc_scatter_body(mem, idx_hbm, vals_hbm, out, idx_v, aux, wlist, rlist,
                     gbuf, gsem, ssem, csem, csem2):
    wid = lax.axis_index("c") * 16 + lax.axis_index("s")
    lo = (8 * ((wid * 12500) >> 5)).astype(jnp.int32)
    hi = (8 * (((wid + 1) * 12500) >> 5)).astype(jnp.int32)
    lane = lax.iota(jnp.int32, L)

    bulk = pltpu.async_copy(mem.at[pl.ds(lo, RPW_MIN)],
                            out.at[pl.ds(lo, RPW_MIN)], csem)
    bulk2 = pltpu.async_copy(mem.at[pl.ds(hi - 8, 8)],
                             out.at[pl.ds(hi - 8, 8)], csem2)

    pltpu.sync_copy(idx_hbm, idx_v)

    neg1 = jnp.full((L,), -1, jnp.int32)

    def init_body(i, carry):
        aux[pl.ds(i * L, L)] = neg1
        return carry

    lax.fori_loop(0, RPW_PAD // L, init_body, 0)

    def fill_body(i, carry):
        v = idx_v[pl.ds(i * L, L)]
        owned = (v >= lo) & (v < hi)
        _, last = plsc.scan_count(v, mask=owned)
        win = last & owned
        local = jnp.where(win, v - lo, 0)
        pos = (i * L + lane).astype(jnp.int32)
        plsc.store_scatter(aux, [local], pos, mask=win)
        return carry

    lax.fori_loop(0, BATCH // L, fill_body, 0)

    def comp_body(c, off):
        a = aux[pl.ds(c * L, L)]
        m = a >= 0
        rows = lo + c * L + lane
        plsc.store_compressed(wlist.at[pl.ds(off, L)], a, mask=m)
        plsc.store_compressed(rlist.at[pl.ds(off, L)], rows, mask=m)
        return off + jnp.sum(m.astype(jnp.int32))

    n = lax.fori_loop(0, RPW_PAD // L, comp_body, jnp.int32(0))

    bulk.wait()
    bulk2.wait()

    @pl.when(n >= L)
    def _():
        nch = (n + L - 1) >> 4

        def dma_body(k, carry):
            o = jnp.minimum(k * L, n - L)
            wv = wlist[pl.ds(o, L)]
            rv = rlist[pl.ds(o, L)]
            pltpu.async_copy(vals_hbm.at[wv], gbuf, gsem).wait()
            pltpu.async_copy(gbuf, out.at[rv], ssem).wait()
            return carry

        lax.fori_loop(0, nch, dma_body, 0)

    @pl.when((n > 0) & (n < L))
    def _():
        wv = wlist[pl.ds(0, L)]
        rv = rlist[pl.ds(0, L)]

        def tail_body(i, carry):
            @pl.when(i < n)
            def _():
                wsc = jnp.max(jnp.where(lane == i, wv, -1))
                rsc = jnp.max(jnp.where(lane == i, rv, -1))
                pltpu.sync_copy(vals_hbm.at[pl.ds(wsc, 1)],
                                gbuf.at[pl.ds(0, 1)])
                pltpu.sync_copy(gbuf.at[pl.ds(0, 1)],
                                out.at[pl.ds(rsc, 1)])
            return carry

        lax.fori_loop(0, L, tail_body, 0)


def kernel(memory, node_idxs, values):
    mem2 = memory.reshape(N_NODES, ROW)
    vals2 = values.reshape(BATCH, ROW)
    idx = node_idxs.astype(jnp.int32)
    out = _build_sc_scatter()(mem2, idx, vals2)
    return out.reshape(memory.shape)


# speedup vs baseline: 31.3326x; 31.3326x over previous
import functools

import jax
import jax.numpy as jnp
from jax import lax
from jax.experimental import pallas as pl
from jax.experimental.pallas import tpu as pltpu
from jax.experimental.pallas import tpu_sc as plsc

N_NODES = 100000
ROW = 512
BATCH = 16384
L = 16
NW = 32
RPW_MIN = 3120
RPW_PAD = 3136
LIST_LEN = RPW_PAD + L


@functools.cache
def _build_sc_scatter():
    mesh = plsc.VectorSubcoreMesh(
        core_axis_name="c", subcore_axis_name="s", num_cores=2, num_subcores=16
    )
    return pl.kernel(
        _sc_scatter_body,
        out_type=(),
        mesh=mesh,
        compiler_params=pltpu.CompilerParams(needs_layout_passes=False),
        scratch_types=[
            pltpu.VMEM((BATCH,), jnp.int32),
            pltpu.VMEM((RPW_PAD,), jnp.int32),
            pltpu.VMEM((LIST_LEN,), jnp.int32),
            pltpu.VMEM((LIST_LEN,), jnp.int32),
            pltpu.VMEM((L, 4, 128), jnp.float32),
            pltpu.SemaphoreType.DMA,
            pltpu.SemaphoreType.DMA,
        ],
    )


def _sc_scatter_body(out, idx_hbm, vals_hbm, idx_v, aux, wlist, rlist,
                     gbuf, gsem, ssem):
    wid = lax.axis_index("c") * 16 + lax.axis_index("s")
    lo = (8 * ((wid * 12500) >> 5)).astype(jnp.int32)
    hi = (8 * (((wid + 1) * 12500) >> 5)).astype(jnp.int32)
    lane = lax.iota(jnp.int32, L)

    pltpu.sync_copy(idx_hbm, idx_v)

    neg1 = jnp.full((L,), -1, jnp.int32)

    def init_body(i, carry):
        aux[pl.ds(i * L, L)] = neg1
        return carry

    lax.fori_loop(0, RPW_PAD // L, init_body, 0)

    def fill_body(i, carry):
        v = idx_v[pl.ds(i * L, L)]
        owned = (v >= lo) & (v < hi)
        _, last = plsc.scan_count(v, mask=owned)
        win = last & owned
        local = jnp.where(win, v - lo, 0)
        pos = (i * L + lane).astype(jnp.int32)
        plsc.store_scatter(aux, [local], pos, mask=win)
        return carry

    lax.fori_loop(0, BATCH // L, fill_body, 0)

    def comp_body(c, off):
        a = aux[pl.ds(c * L, L)]
        m = a >= 0
        rows = lo + c * L + lane
        plsc.store_compressed(wlist.at[pl.ds(off, L)], a, mask=m)
        plsc.store_compressed(rlist.at[pl.ds(off, L)], rows, mask=m)
        return off + jnp.sum(m.astype(jnp.int32))

    n = lax.fori_loop(0, RPW_PAD // L, comp_body, jnp.int32(0))

    @pl.when(n >= L)
    def _():
        nch = (n + L - 1) >> 4

        def dma_body(k, carry):
            o = jnp.minimum(k * L, n - L)
            wv = wlist[pl.ds(o, L)]
            rv = rlist[pl.ds(o, L)]
            pltpu.async_copy(vals_hbm.at[wv], gbuf, gsem).wait()
            pltpu.async_copy(gbuf, out.at[rv], ssem).wait()
            return carry

        lax.fori_loop(0, nch, dma_body, 0)

    @pl.when((n > 0) & (n < L))
    def _():
        wv = wlist[pl.ds(0, L)]
        rv = rlist[pl.ds(0, L)]

        def tail_body(i, carry):
            @pl.when(i < n)
            def _():
                wsc = jnp.max(jnp.where(lane == i, wv, -1))
                rsc = jnp.max(jnp.where(lane == i, rv, -1))
                pltpu.sync_copy(vals_hbm.at[pl.ds(wsc, 1)],
                                gbuf.at[pl.ds(0, 1)])
                pltpu.sync_copy(gbuf.at[pl.ds(0, 1)],
                                out.at[pl.ds(rsc, 1)])
            return carry

        lax.fori_loop(0, L, tail_body, 0)


def kernel(memory, node_idxs, values):
    idx = node_idxs.astype(jnp.int32)
    ref = jax.new_ref(memory)
    _build_sc_scatter()(ref, idx, values)
    return ref[...]


# speedup vs baseline: 35.1775x; 1.1227x over previous
import functools

import jax
import jax.numpy as jnp
from jax import lax
from jax.experimental import pallas as pl
from jax.experimental.pallas import tpu as pltpu
from jax.experimental.pallas import tpu_sc as plsc

N_NODES = 100000
ROW = 512
BATCH = 16384
L = 16
NW = 32
RPW_MIN = 3120
RPW_PAD = 3136
LIST_LEN = RPW_PAD + L


@functools.cache
def _build_sc_scatter():
    mesh = plsc.VectorSubcoreMesh(
        core_axis_name="c", subcore_axis_name="s", num_cores=2, num_subcores=16
    )
    return pl.kernel(
        _sc_scatter_body,
        out_type=(),
        mesh=mesh,
        compiler_params=pltpu.CompilerParams(needs_layout_passes=False),
        scratch_types=[
            pltpu.VMEM((BATCH,), jnp.int32),
            pltpu.VMEM((RPW_PAD,), jnp.int32),
            pltpu.VMEM((LIST_LEN,), jnp.int32),
            pltpu.VMEM((LIST_LEN,), jnp.int32),
            pltpu.VMEM((8, L, 4, 128), jnp.float32),
            pltpu.SemaphoreType.DMA((8,)),
            pltpu.SemaphoreType.DMA((8,)),
        ],
    )


def _sc_scatter_body(out, idx_hbm, vals_hbm, idx_v, aux, wlist, rlist,
                     gbuf, gsem, ssem):
    wid = lax.axis_index("c") * 16 + lax.axis_index("s")
    lo = (8 * ((wid * 12500) >> 5)).astype(jnp.int32)
    hi = (8 * (((wid + 1) * 12500) >> 5)).astype(jnp.int32)
    lane = lax.iota(jnp.int32, L)

    pltpu.sync_copy(idx_hbm, idx_v)

    neg1 = jnp.full((L,), -1, jnp.int32)

    def init_body(i, carry):
        aux[pl.ds(i * L, L)] = neg1
        return carry

    lax.fori_loop(0, RPW_PAD // L, init_body, 0)

    def fill_body(i, carry):
        v = idx_v[pl.ds(i * L, L)]
        owned = (v >= lo) & (v < hi)
        _, last = plsc.scan_count(v, mask=owned)
        win = last & owned
        local = jnp.where(win, v - lo, 0)
        pos = (i * L + lane).astype(jnp.int32)
        plsc.store_scatter(aux, [local], pos, mask=win)
        return carry

    lax.fori_loop(0, BATCH // L, fill_body, 0)

    def comp_body(c, off):
        a = aux[pl.ds(c * L, L)]
        m = a >= 0
        rows = lo + c * L + lane
        plsc.store_compressed(wlist.at[pl.ds(off, L)], a, mask=m)
        plsc.store_compressed(rlist.at[pl.ds(off, L)], rows, mask=m)
        return off + jnp.sum(m.astype(jnp.int32))

    n = lax.fori_loop(0, RPW_PAD // L, comp_body, jnp.int32(0))

    @pl.when(n >= L)
    def _():
        nch = (n + L - 1) >> 4
        G = 8

        def grp_body(g, carry):
            base = g * G

            def fire(j):
                o = jnp.minimum((base + j) * L, n - L)
                wv = wlist[pl.ds(o, L)]
                return pltpu.async_copy(vals_hbm.at[wv], gbuf.at[j],
                                        gsem.at[j])

            def drain(j):
                o = jnp.minimum((base + j) * L, n - L)
                rv = rlist[pl.ds(o, L)]
                return pltpu.async_copy(gbuf.at[j], out.at[rv], ssem.at[j])

            for j in range(G):
                @pl.when(base + j < nch)
                def _(j=j):
                    fire(j)
            for j in range(G):
                @pl.when(base + j < nch)
                def _(j=j):
                    pltpu.make_async_copy(vals_hbm.at[wlist[pl.ds(0, L)]],
                                          gbuf.at[j], gsem.at[j]).wait()
                    drain(j)
            for j in range(G):
                @pl.when(base + j < nch)
                def _(j=j):
                    pltpu.make_async_copy(gbuf.at[j],
                                          out.at[rlist[pl.ds(0, L)]],
                                          ssem.at[j]).wait()
            return carry

        lax.fori_loop(0, (nch + G - 1) >> 3, grp_body, 0)

    @pl.when((n > 0) & (n < L))
    def _():
        wv = wlist[pl.ds(0, L)]
        rv = rlist[pl.ds(0, L)]

        def tail_body(i, carry):
            @pl.when(i < n)
            def _():
                wsc = jnp.max(jnp.where(lane == i, wv, -1))
                rsc = jnp.max(jnp.where(lane == i, rv, -1))
                pltpu.sync_copy(vals_hbm.at[pl.ds(wsc, 1)],
                                gbuf.at[0, pl.ds(0, 1)])
                pltpu.sync_copy(gbuf.at[0, pl.ds(0, 1)],
                                out.at[pl.ds(rsc, 1)])
            return carry

        lax.fori_loop(0, L, tail_body, 0)


def kernel(memory, node_idxs, values):
    idx = node_idxs.astype(jnp.int32)
    ref = jax.new_ref(memory)
    _build_sc_scatter()(ref, idx, values)
    return ref[...]
